# baseline (speedup 1.0000x reference)
import functools

import jax
import jax.numpy as jnp
from jax import lax
from jax.experimental import pallas as pl
from jax.experimental.pallas import tpu as pltpu
from jax.experimental.pallas import tpu_sc as plsc

_TEMP = 0.07
_K = 5
_LANES = 16
_NCORES = 2
_NSUB = 16
_NW = _NCORES * _NSUB
_LO = 512


def _vrsqrt(q):
    i = plsc.bitcast(q, jnp.int32)
    i = jnp.full((_LANES,), 0x5F3759DF, jnp.int32) - lax.shift_right_logical(
        i, jnp.full((_LANES,), 1, jnp.int32))
    y = plsc.bitcast(i, jnp.float32)
    half_q = 0.5 * q
    for _ in range(3):
        y = y * (1.5 - half_q * y * y)
    return y


def _sc_mine_hi(user_emb, poolT, lo, interpret=False):
    B, D = user_emb.shape
    P = poolT.shape[0]
    nhi = B - lo
    rows_w = nhi // _NW
    dchunks = D // _LANES
    pchunks = (P + _LANES - 1) // _LANES

    mesh = plsc.VectorSubcoreMesh(core_axis_name="c", subcore_axis_name="s")
    grp = 8

    @functools.partial(
        pl.kernel,
        out_type=jax.ShapeDtypeStruct((nhi, _LANES), jnp.float32),
        mesh=mesh,
        interpret=interpret,
        compiler_params=pltpu.CompilerParams(needs_layout_passes=False),
        cost_estimate=pl.CostEstimate(
            flops=4 * nhi * P * D,
            bytes_accessed=nhi * P * D * 4,
            transcendentals=0),
        scratch_types=[
            pltpu.VMEM((8, D), jnp.float32),
            pltpu.VMEM((P, 8, D), jnp.float32),
            pltpu.VMEM((P, 8, D), jnp.float32),
            pltpu.VMEM((rows_w, _LANES), jnp.float32),
            pltpu.SemaphoreType.DMA,
            pltpu.SemaphoreType.DMA,
        ],
    )
    def k(u_hbm, pool_hbm, out_hbm, u_v, row_v0, row_v1, o_v, sem0, sem1):
        wid = lax.axis_index("s") * _NCORES + lax.axis_index("c")
        base = wid * rows_w

        iota = lax.iota(jnp.int32, _LANES)
        lane_masks = [iota == jj for jj in range(_LANES)]
        neg_fill = jnp.full((_LANES,), -1e30, jnp.float32)
        ones = jnp.full((_LANES,), 1.0, jnp.float32)

        def fetch(g, buf, sem):
            return pltpu.make_async_copy(
                pool_hbm.at[:, pl.ds(lo + base + g * grp, grp)], buf, sem)

        fetch(0, row_v0, sem0).start()
        fetch(1, row_v1, sem1).start()

        def do_row(r, rsub, row_v):
            uk = [u_v[rsub, pl.ds(kk * _LANES, _LANES)] for kk in range(dchunks)]
            qu01 = uk[0] * uk[0] + uk[1] * uk[1]
            qu23 = uk[2] * uk[2] + uk[3] * uk[3]
            qu45 = uk[4] * uk[4] + uk[5] * uk[5]
            qu67 = uk[6] * uk[6] + uk[7] * uk[7]
            qus = jnp.maximum(jnp.sum((qu01 + qu23) + (qu45 + qu67)), 1e-24)
            ru = jnp.max(_vrsqrt(jnp.full((_LANES,), qus, jnp.float32)))
            uk = [x * ru for x in uk]
            dcs = [neg_fill] * pchunks
            qcs = [ones] * pchunks
            for j in range(P):
                nk = [row_v[j, rsub, pl.ds(kk * _LANES, _LANES)]
                      for kk in range(dchunks)]
                da = uk[0] * nk[0]
                db = uk[1] * nk[1]
                qa = nk[0] * nk[0]
                qb = nk[1] * nk[1]
                for kk in range(2, dchunks, 2):
                    da = da + uk[kk] * nk[kk]
                    db = db + uk[kk + 1] * nk[kk + 1]
                    qa = qa + nk[kk] * nk[kk]
                    qb = qb + nk[kk + 1] * nk[kk + 1]
                c, l = divmod(j, _LANES)
                dcs[c] = jnp.where(lane_masks[l], jnp.sum(da + db), dcs[c])
                qcs[c] = jnp.where(lane_masks[l], jnp.sum(qa + qb), qcs[c])
            vs = [dcs[c] * _vrsqrt(jnp.maximum(qcs[c], 1e-24))
                  for c in range(pchunks)]
            t = neg_fill
            for i in range(_K):
                m = [jnp.max(v) for v in vs]
                g = jnp.maximum(jnp.maximum(m[0], m[1]),
                                jnp.maximum(m[2], m[3]))
                t = jnp.where(lane_masks[i], g, t)
                taken = jnp.zeros((), jnp.bool_)
                nvs = []
                for c in range(pchunks):
                    hit = m[c] == g
                    take = jnp.logical_and(hit, jnp.logical_not(taken))
                    taken = jnp.logical_or(taken, hit)
                    lane = jnp.min(jnp.where(vs[c] == g, iota, 99))
                    rm = jnp.logical_and(iota == lane, take)
                    nvs.append(jnp.where(rm, -3.0e38, vs[c]))
                vs = nvs
            o_v[r] = t

        ngroups = rows_w // grp

        def do_group(g, row_v, sem):
            pltpu.sync_copy(u_hbm.at[pl.ds(lo + base + g * grp, grp)], u_v)
            fetch(g, row_v, sem).wait()

            def row_body(rsub, carry):
                do_row(g * grp + rsub, rsub, row_v)
                return carry

            lax.fori_loop(0, grp, row_body, 0)

        def body(h, carry):
            g0 = 2 * h
            do_group(g0, row_v0, sem0)

            @pl.when(g0 + 2 < ngroups)
            def _():
                fetch(g0 + 2, row_v0, sem0).start()

            g1 = g0 + 1
            do_group(g1, row_v1, sem1)

            @pl.when(g1 + 2 < ngroups)
            def _():
                fetch(g1 + 2, row_v1, sem1).start()

            return carry

        lax.fori_loop(0, ngroups // 2, body, 0)
        pltpu.sync_copy(o_v, out_hbm.at[pl.ds(base, rows_w)])

    return k(user_emb, poolT)


def _tc_sims(user_emb, poolT, interpret=False):
    P, B, D = poolT.shape
    lo = _LO
    BM = 256
    PP = 64

    def body(u_ref, pool_ref, o_ref):
        u = u_ref[...]
        pn = pool_ref[...]
        un = u * lax.rsqrt(jnp.maximum(jnp.sum(u * u, axis=1, keepdims=True), 1e-24))
        d = jnp.sum(un[None, :, :] * pn, axis=2)
        q = jnp.maximum(jnp.sum(pn * pn, axis=2), 1e-24)
        s = d * lax.rsqrt(q)
        pad = jnp.full((PP - P, BM), -1e30, jnp.float32)
        o_ref[...] = jnp.concatenate([s, pad], axis=0)

    return pl.pallas_call(
        body,
        grid=(lo // BM,),
        in_specs=[pl.BlockSpec((BM, D), lambda i: (i, 0)),
                  pl.BlockSpec((P, BM, D), lambda i: (0, i, 0))],
        out_specs=pl.BlockSpec((PP, BM), lambda i: (0, i)),
        out_shape=jax.ShapeDtypeStruct((PP, lo), jnp.float32),
        interpret=interpret,
    )(user_emb, poolT)


def _sc_topk(sims, gate, interpret=False):
    PP, B = sims.shape
    rows_w = B // _NW
    pchunks = PP // _LANES

    mesh = plsc.VectorSubcoreMesh(core_axis_name="c", subcore_axis_name="s")

    @functools.partial(
        pl.kernel,
        out_type=jax.ShapeDtypeStruct((B, _LANES), jnp.float32),
        mesh=mesh,
        interpret=interpret,
        compiler_params=pltpu.CompilerParams(needs_layout_passes=False),
        scratch_types=[
            pltpu.VMEM((PP, 128), jnp.float32),
            pltpu.VMEM((rows_w, _LANES), jnp.float32),
        ],
    )
    def k(s_hbm, gate_hbm, out_hbm, s_v, o_v):
        del gate_hbm
        wid = lax.axis_index("s") * _NCORES + lax.axis_index("c")
        base = wid * rows_w
        tile = base // 128
        off = base - tile * 128
        pltpu.sync_copy(s_hbm.at[:, pl.ds(tile * 128, 128)], s_v)

        iota = lax.iota(jnp.int32, _LANES)
        lane_masks = [iota == jj for jj in range(_LANES)]
        neg_fill = jnp.full((_LANES,), -1e30, jnp.float32)

        def body(r, carry):
            rcol = jnp.full((_LANES,), 0, jnp.int32) + off + r
            vs = [plsc.load_gather(s_v, [iota + c * _LANES, rcol])
                  for c in range(pchunks)]
            t = neg_fill
            for i in range(_K):
                m = [jnp.max(v) for v in vs]
                g = jnp.maximum(jnp.maximum(m[0], m[1]),
                                jnp.maximum(m[2], m[3]))
                t = jnp.where(lane_masks[i], g, t)
                taken = jnp.zeros((), jnp.bool_)
                nvs = []
                for c in range(pchunks):
                    hit = m[c] == g
                    take = jnp.logical_and(hit, jnp.logical_not(taken))
                    taken = jnp.logical_or(taken, hit)
                    lane = jnp.min(jnp.where(vs[c] == g, iota, 99))
                    rm = jnp.logical_and(iota == lane, take)
                    nvs.append(jnp.where(rm, -3.0e38, vs[c]))
                vs = nvs
            o_v[r] = t
            return carry

        lax.fori_loop(0, rows_w, body, 0)
        pltpu.sync_copy(o_v, out_hbm.at[pl.ds(base, rows_w)])

    return k(sims, gate)


def _tc_stats(user_emb, pos_emb, interpret=False):
    B, D = user_emb.shape
    BM = 256

    def body(u_ref, p_ref, o_ref):
        i = pl.program_id(0)
        u = u_ref[...]
        p = p_ref[...]
        un = u * lax.rsqrt(jnp.maximum(jnp.sum(u * u, axis=1, keepdims=True), 1e-24))
        pn = p * lax.rsqrt(jnp.maximum(jnp.sum(p * p, axis=1, keepdims=True), 1e-24))
        s = lax.dot_general(
            un, pn, (((1,), (1,)), ((), ())),
            preferred_element_type=jnp.float32,
            precision=lax.Precision.HIGHEST) * (1.0 / _TEMP)
        rows = i * BM + lax.broadcasted_iota(jnp.int32, (BM, B), 0)
        cols = lax.broadcasted_iota(jnp.int32, (BM, B), 1)
        diag = rows == cols
        se = jnp.sum(jnp.where(diag, 0.0, jnp.exp(s)), axis=1)
        rmax = jnp.max(jnp.where(diag, -3.0e38, s), axis=1)
        pos = jnp.sum(jnp.where(diag, s, 0.0), axis=1)
        o_ref[...] = jnp.concatenate(
            [pos[:, None], se[:, None], rmax[:, None],
             jnp.zeros((BM, 5), jnp.float32)], axis=1)

    return pl.pallas_call(
        body,
        grid=(B // BM,),
        in_specs=[pl.BlockSpec((BM, D), lambda i: (i, 0)),
                  pl.BlockSpec((B, D), lambda i: (0, 0))],
        out_specs=pl.BlockSpec((BM, 8), lambda i: (i, 0)),
        out_shape=jax.ShapeDtypeStruct((B, 8), jnp.float32),
        interpret=interpret,
    )(user_emb, pos_emb)


def _tc_combine(stats, top_lo, top_hi, interpret=False):
    def body(st_ref, lo_ref, hi_ref, o_ref):
        pos = st_ref[:, 0:1]
        se = st_ref[:, 1:2]
        rmax = st_ref[:, 2:3]
        hn_lo = lo_ref[...] * (1.0 / _TEMP)
        hn_hi = hi_ref[...] * (1.0 / _TEMP)
        he = jnp.concatenate(
            [jnp.sum(jnp.exp(hn_lo), axis=1, keepdims=True),
             jnp.sum(jnp.exp(hn_hi), axis=1, keepdims=True)], axis=0)
        hmax = jnp.concatenate([hn_lo[:, 0:1], hn_hi[:, 0:1]], axis=0)
        lse = jnp.log(se + jnp.exp(pos) + he)
        o_ref[0] = jnp.mean(lse - pos)
        maxo = jnp.maximum(rmax, hmax)
        o_ref[1] = jnp.mean((pos >= maxo).astype(jnp.float32))
        o_ref[2] = jnp.mean(pos)
        o_ref[3] = jnp.mean(hmax)

    return pl.pallas_call(
        body,
        out_specs=pl.BlockSpec(memory_space=pltpu.SMEM),
        out_shape=jax.ShapeDtypeStruct((4,), jnp.float32),
        interpret=interpret,
    )(stats, top_lo, top_hi)


def kernel(user_emb, pos_emb, neg_emb_pool):
    poolT = jnp.transpose(neg_emb_pool, (1, 0, 2))
    top_hi = _sc_mine_hi(user_emb, poolT, _LO)
    sims = _tc_sims(user_emb, poolT)
    top_lo = _sc_topk(sims, lax.slice(top_hi, (0, 0), (8, 16)))
    stats = _tc_stats(user_emb, pos_emb)
    out = _tc_combine(stats, top_lo, top_hi)
    return (out[0], out[1], out[2], out[3])


# speedup vs baseline: 1.5185x; 1.5185x over previous
import functools

import jax
import jax.numpy as jnp
from jax import lax
from jax.experimental import pallas as pl
from jax.experimental.pallas import tpu as pltpu
from jax.experimental.pallas import tpu_sc as plsc

_TEMP = 0.07
_K = 5
_LANES = 16
_NCORES = 2
_NSUB = 16
_NW = _NCORES * _NSUB
_LO = 512


def _vrsqrt(q):
    i = plsc.bitcast(q, jnp.int32)
    i = jnp.full((_LANES,), 0x5F3759DF, jnp.int32) - lax.shift_right_logical(
        i, jnp.full((_LANES,), 1, jnp.int32))
    y = plsc.bitcast(i, jnp.float32)
    half_q = 0.5 * q
    for _ in range(3):
        y = y * (1.5 - half_q * y * y)
    return y


def _sc_mine_hi(user_emb, poolT, lo, interpret=False):
    B, D = user_emb.shape
    P = poolT.shape[0]
    nhi = B - lo
    rows_w = nhi // _NW
    dchunks = D // _LANES
    pchunks = (P + _LANES - 1) // _LANES

    mesh = plsc.VectorSubcoreMesh(core_axis_name="c", subcore_axis_name="s")
    grp = 8

    @functools.partial(
        pl.kernel,
        out_type=jax.ShapeDtypeStruct((nhi, _LANES), jnp.float32),
        mesh=mesh,
        interpret=interpret,
        compiler_params=pltpu.CompilerParams(needs_layout_passes=False),
        cost_estimate=pl.CostEstimate(
            flops=4 * nhi * P * D,
            bytes_accessed=nhi * P * D * 4,
            transcendentals=0),
        scratch_types=[
            pltpu.VMEM((8, D), jnp.float32),
            pltpu.VMEM((P, 8, D), jnp.float32),
            pltpu.VMEM((P, 8, D), jnp.float32),
            pltpu.VMEM((rows_w, _LANES), jnp.float32),
            pltpu.SemaphoreType.DMA,
            pltpu.SemaphoreType.DMA,
        ],
    )
    def k(u_hbm, pool_hbm, out_hbm, u_v, row_v0, row_v1, o_v, sem0, sem1):
        wid = lax.axis_index("s") * _NCORES + lax.axis_index("c")
        base = wid * rows_w

        iota = lax.iota(jnp.int32, _LANES)
        lane_masks = [iota == jj for jj in range(_LANES)]
        neg_fill = jnp.full((_LANES,), -1e30, jnp.float32)
        ones = jnp.full((_LANES,), 1.0, jnp.float32)

        def fetch(g, buf, sem):
            return pltpu.make_async_copy(
                pool_hbm.at[:, pl.ds(lo + base + g * grp, grp)], buf, sem)

        fetch(0, row_v0, sem0).start()
        fetch(1, row_v1, sem1).start()

        def do_row(r, rsub, row_v):
            uk = [u_v[rsub, pl.ds(kk * _LANES, _LANES)] for kk in range(dchunks)]
            qu01 = uk[0] * uk[0] + uk[1] * uk[1]
            qu23 = uk[2] * uk[2] + uk[3] * uk[3]
            qu45 = uk[4] * uk[4] + uk[5] * uk[5]
            qu67 = uk[6] * uk[6] + uk[7] * uk[7]
            qus = jnp.maximum(jnp.sum((qu01 + qu23) + (qu45 + qu67)), 1e-24)
            ru = jnp.max(_vrsqrt(jnp.full((_LANES,), qus, jnp.float32)))
            uk = [x * ru for x in uk]
            dcs = [neg_fill] * pchunks
            qcs = [ones] * pchunks
            for j in range(P):
                nk = [row_v[j, rsub, pl.ds(kk * _LANES, _LANES)]
                      for kk in range(dchunks)]
                da = uk[0] * nk[0]
                db = uk[1] * nk[1]
                qa = nk[0] * nk[0]
                qb = nk[1] * nk[1]
                for kk in range(2, dchunks, 2):
                    da = da + uk[kk] * nk[kk]
                    db = db + uk[kk + 1] * nk[kk + 1]
                    qa = qa + nk[kk] * nk[kk]
                    qb = qb + nk[kk + 1] * nk[kk + 1]
                c, l = divmod(j, _LANES)
                dcs[c] = jnp.where(lane_masks[l], jnp.sum(da + db), dcs[c])
                qcs[c] = jnp.where(lane_masks[l], jnp.sum(qa + qb), qcs[c])
            vs = [dcs[c] * _vrsqrt(jnp.maximum(qcs[c], 1e-24))
                  for c in range(pchunks)]
            t = neg_fill
            for i in range(_K):
                m = [jnp.max(v) for v in vs]
                g = jnp.maximum(jnp.maximum(m[0], m[1]),
                                jnp.maximum(m[2], m[3]))
                t = jnp.where(lane_masks[i], g, t)
                taken = jnp.zeros((), jnp.bool_)
                nvs = []
                for c in range(pchunks):
                    hit = m[c] == g
                    take = jnp.logical_and(hit, jnp.logical_not(taken))
                    taken = jnp.logical_or(taken, hit)
                    lane = jnp.min(jnp.where(vs[c] == g, iota, 99))
                    rm = jnp.logical_and(iota == lane, take)
                    nvs.append(jnp.where(rm, -3.0e38, vs[c]))
                vs = nvs
            o_v[r] = t

        ngroups = rows_w // grp

        def do_group(g, row_v, sem):
            pltpu.sync_copy(u_hbm.at[pl.ds(lo + base + g * grp, grp)], u_v)
            fetch(g, row_v, sem).wait()

            def row_body(rsub, carry):
                do_row(g * grp + rsub, rsub, row_v)
                return carry

            lax.fori_loop(0, grp, row_body, 0)

        def body(h, carry):
            g0 = 2 * h
            do_group(g0, row_v0, sem0)

            @pl.when(g0 + 2 < ngroups)
            def _():
                fetch(g0 + 2, row_v0, sem0).start()

            g1 = g0 + 1
            do_group(g1, row_v1, sem1)

            @pl.when(g1 + 2 < ngroups)
            def _():
                fetch(g1 + 2, row_v1, sem1).start()

            return carry

        lax.fori_loop(0, ngroups // 2, body, 0)
        pltpu.sync_copy(o_v, out_hbm.at[pl.ds(base, rows_w)])

    return k(user_emb, poolT)


def _tc_sims(user_emb, poolT, interpret=False):
    P, B, D = poolT.shape
    lo = _LO
    BM = 256
    PP = 64

    def body(u_ref, pool_ref, o_ref):
        u = u_ref[...]
        pn = pool_ref[...]
        un = u * lax.rsqrt(jnp.maximum(jnp.sum(u * u, axis=1, keepdims=True), 1e-24))
        d = jnp.sum(un[None, :, :] * pn, axis=2)
        q = jnp.maximum(jnp.sum(pn * pn, axis=2), 1e-24)
        s = d * lax.rsqrt(q)
        pad = jnp.full((PP - P, BM), -1e30, jnp.float32)
        o_ref[...] = jnp.concatenate([s, pad], axis=0)

    return pl.pallas_call(
        body,
        grid=(lo // BM,),
        in_specs=[pl.BlockSpec((BM, D), lambda i: (i, 0)),
                  pl.BlockSpec((P, BM, D), lambda i: (0, i, 0))],
        out_specs=pl.BlockSpec((PP, BM), lambda i: (0, i)),
        out_shape=jax.ShapeDtypeStruct((PP, lo), jnp.float32),
        interpret=interpret,
    )(user_emb, poolT)


def _sc_topk(sims, gate, interpret=False):
    PP, B = sims.shape
    rows_w = B // _NW
    pchunks = PP // _LANES

    mesh = plsc.VectorSubcoreMesh(core_axis_name="c", subcore_axis_name="s")

    @functools.partial(
        pl.kernel,
        out_type=jax.ShapeDtypeStruct((B, _LANES), jnp.float32),
        mesh=mesh,
        interpret=interpret,
        compiler_params=pltpu.CompilerParams(needs_layout_passes=False),
        scratch_types=[
            pltpu.VMEM((PP, 128), jnp.float32),
            pltpu.VMEM((rows_w, _LANES), jnp.float32),
        ],
    )
    def k(s_hbm, gate_hbm, out_hbm, s_v, o_v):
        del gate_hbm
        wid = lax.axis_index("s") * _NCORES + lax.axis_index("c")
        base = wid * rows_w
        tile = base // 128
        off = base - tile * 128
        pltpu.sync_copy(s_hbm.at[:, pl.ds(tile * 128, 128)], s_v)

        iota = lax.iota(jnp.int32, _LANES)
        lane_masks = [iota == jj for jj in range(_LANES)]
        neg_fill = jnp.full((_LANES,), -1e30, jnp.float32)

        def body(r, carry):
            rcol = jnp.full((_LANES,), 0, jnp.int32) + off + r
            vs = [plsc.load_gather(s_v, [iota + c * _LANES, rcol])
                  for c in range(pchunks)]
            t = neg_fill
            for i in range(_K):
                m = [jnp.max(v) for v in vs]
                g = jnp.maximum(jnp.maximum(m[0], m[1]),
                                jnp.maximum(m[2], m[3]))
                t = jnp.where(lane_masks[i], g, t)
                taken = jnp.zeros((), jnp.bool_)
                nvs = []
                for c in range(pchunks):
                    hit = m[c] == g
                    take = jnp.logical_and(hit, jnp.logical_not(taken))
                    taken = jnp.logical_or(taken, hit)
                    lane = jnp.min(jnp.where(vs[c] == g, iota, 99))
                    rm = jnp.logical_and(iota == lane, take)
                    nvs.append(jnp.where(rm, -3.0e38, vs[c]))
                vs = nvs
            o_v[r] = t
            return carry

        lax.fori_loop(0, rows_w, body, 0)
        pltpu.sync_copy(o_v, out_hbm.at[pl.ds(base, rows_w)])

    return k(sims, gate)


def _tc_stats(user_emb, pos_emb, interpret=False):
    B, D = user_emb.shape
    BM = 256

    def body(u_ref, p_ref, o_ref):
        i = pl.program_id(0)
        u = u_ref[...]
        p = p_ref[...]
        un = u * lax.rsqrt(jnp.maximum(jnp.sum(u * u, axis=1, keepdims=True), 1e-24))
        pn = p * lax.rsqrt(jnp.maximum(jnp.sum(p * p, axis=1, keepdims=True), 1e-24))
        s = lax.dot_general(
            un, pn, (((1,), (1,)), ((), ())),
            preferred_element_type=jnp.float32,
            precision=lax.Precision.HIGHEST) * (1.0 / _TEMP)
        rows = i * BM + lax.broadcasted_iota(jnp.int32, (BM, B), 0)
        cols = lax.broadcasted_iota(jnp.int32, (BM, B), 1)
        diag = rows == cols
        se = jnp.sum(jnp.where(diag, 0.0, jnp.exp(s)), axis=1)
        rmax = jnp.max(jnp.where(diag, -3.0e38, s), axis=1)
        pos = jnp.sum(jnp.where(diag, s, 0.0), axis=1)
        o_ref[...] = jnp.concatenate(
            [pos[:, None], se[:, None], rmax[:, None],
             jnp.zeros((BM, 5), jnp.float32)], axis=1)

    return pl.pallas_call(
        body,
        grid=(B // BM,),
        in_specs=[pl.BlockSpec((BM, D), lambda i: (i, 0)),
                  pl.BlockSpec((B, D), lambda i: (0, 0))],
        out_specs=pl.BlockSpec((BM, 8), lambda i: (i, 0)),
        out_shape=jax.ShapeDtypeStruct((B, 8), jnp.float32),
        interpret=interpret,
    )(user_emb, pos_emb)


def _tc_combine(stats, top_lo, top_hi, interpret=False):
    def body(st_ref, lo_ref, hi_ref, o_ref):
        pos = st_ref[:, 0:1]
        se = st_ref[:, 1:2]
        rmax = st_ref[:, 2:3]
        hn_lo = lo_ref[...] * (1.0 / _TEMP)
        hn_hi = hi_ref[...] * (1.0 / _TEMP)
        he = jnp.concatenate(
            [jnp.sum(jnp.exp(hn_lo), axis=1, keepdims=True),
             jnp.sum(jnp.exp(hn_hi), axis=1, keepdims=True)], axis=0)
        hmax = jnp.concatenate([hn_lo[:, 0:1], hn_hi[:, 0:1]], axis=0)
        lse = jnp.log(se + jnp.exp(pos) + he)
        o_ref[0] = jnp.mean(lse - pos)
        maxo = jnp.maximum(rmax, hmax)
        o_ref[1] = jnp.mean((pos >= maxo).astype(jnp.float32))
        o_ref[2] = jnp.mean(pos)
        o_ref[3] = jnp.mean(hmax)

    return pl.pallas_call(
        body,
        out_specs=pl.BlockSpec(memory_space=pltpu.SMEM),
        out_shape=jax.ShapeDtypeStruct((4,), jnp.float32),
        interpret=interpret,
    )(stats, top_lo, top_hi)


def kernel(user_emb, pos_emb, neg_emb_pool):
    poolT = jnp.transpose(neg_emb_pool, (1, 0, 2))
    top_hi = _sc_mine_hi(user_emb, poolT, _LO)
    sims = _tc_sims(user_emb, poolT)
    top_lo = _sc_topk(sims, lax.slice(sims, (0, 0), (8, 16)))
    stats = _tc_stats(user_emb, pos_emb)
    out = _tc_combine(stats, top_lo, top_hi)
    return (out[0], out[1], out[2], out[3])


# speedup vs baseline: 1.5576x; 1.0257x over previous
import functools

import jax
import jax.numpy as jnp
from jax import lax
from jax.experimental import pallas as pl
from jax.experimental.pallas import tpu as pltpu
from jax.experimental.pallas import tpu_sc as plsc

_TEMP = 0.07
_K = 5
_LANES = 16
_NCORES = 2
_NSUB = 16
_NW = _NCORES * _NSUB
_LO = 1024


def _vrsqrt(q):
    i = plsc.bitcast(q, jnp.int32)
    i = jnp.full((_LANES,), 0x5F3759DF, jnp.int32) - lax.shift_right_logical(
        i, jnp.full((_LANES,), 1, jnp.int32))
    y = plsc.bitcast(i, jnp.float32)
    half_q = 0.5 * q
    for _ in range(3):
        y = y * (1.5 - half_q * y * y)
    return y


def _sc_mine_hi(user_emb, poolT, lo, interpret=False):
    B, D = user_emb.shape
    P = poolT.shape[0]
    nhi = B - lo
    rows_w = nhi // _NW
    dchunks = D // _LANES
    pchunks = (P + _LANES - 1) // _LANES

    mesh = plsc.VectorSubcoreMesh(core_axis_name="c", subcore_axis_name="s")
    grp = 8

    @functools.partial(
        pl.kernel,
        out_type=jax.ShapeDtypeStruct((nhi, _LANES), jnp.float32),
        mesh=mesh,
        interpret=interpret,
        compiler_params=pltpu.CompilerParams(needs_layout_passes=False),
        cost_estimate=pl.CostEstimate(
            flops=4 * nhi * P * D,
            bytes_accessed=nhi * P * D * 4,
            transcendentals=0),
        scratch_types=[
            pltpu.VMEM((8, D), jnp.float32),
            pltpu.VMEM((P, 8, D), jnp.float32),
            pltpu.VMEM((P, 8, D), jnp.float32),
            pltpu.VMEM((rows_w, _LANES), jnp.float32),
            pltpu.SemaphoreType.DMA,
            pltpu.SemaphoreType.DMA,
        ],
    )
    def k(u_hbm, pool_hbm, out_hbm, u_v, row_v0, row_v1, o_v, sem0, sem1):
        wid = lax.axis_index("s") * _NCORES + lax.axis_index("c")
        base = wid * rows_w

        iota = lax.iota(jnp.int32, _LANES)
        lane_masks = [iota == jj for jj in range(_LANES)]
        neg_fill = jnp.full((_LANES,), -1e30, jnp.float32)
        ones = jnp.full((_LANES,), 1.0, jnp.float32)

        def fetch(g, buf, sem):
            return pltpu.make_async_copy(
                pool_hbm.at[:, pl.ds(lo + base + g * grp, grp)], buf, sem)

        fetch(0, row_v0, sem0).start()
        fetch(1, row_v1, sem1).start()

        def do_row(r, rsub, row_v):
            uk = [u_v[rsub, pl.ds(kk * _LANES, _LANES)] for kk in range(dchunks)]
            qu01 = uk[0] * uk[0] + uk[1] * uk[1]
            qu23 = uk[2] * uk[2] + uk[3] * uk[3]
            qu45 = uk[4] * uk[4] + uk[5] * uk[5]
            qu67 = uk[6] * uk[6] + uk[7] * uk[7]
            qus = jnp.maximum(jnp.sum((qu01 + qu23) + (qu45 + qu67)), 1e-24)
            ru = jnp.max(_vrsqrt(jnp.full((_LANES,), qus, jnp.float32)))
            uk = [x * ru for x in uk]
            dcs = [neg_fill] * pchunks
            qcs = [ones] * pchunks
            for j in range(P):
                nk = [row_v[j, rsub, pl.ds(kk * _LANES, _LANES)]
                      for kk in range(dchunks)]
                da = uk[0] * nk[0]
                db = uk[1] * nk[1]
                qa = nk[0] * nk[0]
                qb = nk[1] * nk[1]
                for kk in range(2, dchunks, 2):
                    da = da + uk[kk] * nk[kk]
                    db = db + uk[kk + 1] * nk[kk + 1]
                    qa = qa + nk[kk] * nk[kk]
                    qb = qb + nk[kk + 1] * nk[kk + 1]
                c, l = divmod(j, _LANES)
                dcs[c] = jnp.where(lane_masks[l], jnp.sum(da + db), dcs[c])
                qcs[c] = jnp.where(lane_masks[l], jnp.sum(qa + qb), qcs[c])
            vs = [dcs[c] * _vrsqrt(jnp.maximum(qcs[c], 1e-24))
                  for c in range(pchunks)]
            t = neg_fill
            for i in range(_K):
                m = [jnp.max(v) for v in vs]
                g = jnp.maximum(jnp.maximum(m[0], m[1]),
                                jnp.maximum(m[2], m[3]))
                t = jnp.where(lane_masks[i], g, t)
                taken = jnp.zeros((), jnp.bool_)
                nvs = []
                for c in range(pchunks):
                    hit = m[c] == g
                    take = jnp.logical_and(hit, jnp.logical_not(taken))
                    taken = jnp.logical_or(taken, hit)
                    lane = jnp.min(jnp.where(vs[c] == g, iota, 99))
                    rm = jnp.logical_and(iota == lane, take)
                    nvs.append(jnp.where(rm, -3.0e38, vs[c]))
                vs = nvs
            o_v[r] = t

        ngroups = rows_w // grp

        def do_group(g, row_v, sem):
            pltpu.sync_copy(u_hbm.at[pl.ds(lo + base + g * grp, grp)], u_v)
            fetch(g, row_v, sem).wait()

            def row_body(rsub, carry):
                do_row(g * grp + rsub, rsub, row_v)
                return carry

            lax.fori_loop(0, grp, row_body, 0)

        def body(h, carry):
            g0 = 2 * h
            do_group(g0, row_v0, sem0)

            @pl.when(g0 + 2 < ngroups)
            def _():
                fetch(g0 + 2, row_v0, sem0).start()

            g1 = g0 + 1
            do_group(g1, row_v1, sem1)

            @pl.when(g1 + 2 < ngroups)
            def _():
                fetch(g1 + 2, row_v1, sem1).start()

            return carry

        lax.fori_loop(0, ngroups // 2, body, 0)
        pltpu.sync_copy(o_v, out_hbm.at[pl.ds(base, rows_w)])

    return k(user_emb, poolT)


def _tc_sims(user_emb, poolT, interpret=False):
    P, B, D = poolT.shape
    lo = _LO
    BM = 256
    PP = 64

    def body(u_ref, pool_ref, o_ref):
        u = u_ref[...]
        pn = pool_ref[...]
        un = u * lax.rsqrt(jnp.maximum(jnp.sum(u * u, axis=1, keepdims=True), 1e-24))
        d = jnp.sum(un[None, :, :] * pn, axis=2)
        q = jnp.maximum(jnp.sum(pn * pn, axis=2), 1e-24)
        s = d * lax.rsqrt(q)
        pad = jnp.full((PP - P, BM), -1e30, jnp.float32)
        o_ref[...] = jnp.concatenate([s, pad], axis=0)

    return pl.pallas_call(
        body,
        grid=(lo // BM,),
        in_specs=[pl.BlockSpec((BM, D), lambda i: (i, 0)),
                  pl.BlockSpec((P, BM, D), lambda i: (0, i, 0))],
        out_specs=pl.BlockSpec((PP, BM), lambda i: (0, i)),
        out_shape=jax.ShapeDtypeStruct((PP, lo), jnp.float32),
        interpret=interpret,
    )(user_emb, poolT)


def _sc_topk(sims, gate, interpret=False):
    PP, B = sims.shape
    rows_w = B // _NW
    pchunks = PP // _LANES

    mesh = plsc.VectorSubcoreMesh(core_axis_name="c", subcore_axis_name="s")

    @functools.partial(
        pl.kernel,
        out_type=jax.ShapeDtypeStruct((B, _LANES), jnp.float32),
        mesh=mesh,
        interpret=interpret,
        compiler_params=pltpu.CompilerParams(needs_layout_passes=False),
        scratch_types=[
            pltpu.VMEM((PP, 128), jnp.float32),
            pltpu.VMEM((rows_w, _LANES), jnp.float32),
        ],
    )
    def k(s_hbm, gate_hbm, out_hbm, s_v, o_v):
        del gate_hbm
        wid = lax.axis_index("s") * _NCORES + lax.axis_index("c")
        base = wid * rows_w
        tile = base // 128
        off = base - tile * 128
        pltpu.sync_copy(s_hbm.at[:, pl.ds(tile * 128, 128)], s_v)

        iota = lax.iota(jnp.int32, _LANES)
        lane_masks = [iota == jj for jj in range(_LANES)]
        neg_fill = jnp.full((_LANES,), -1e30, jnp.float32)

        def body(r, carry):
            rcol = jnp.full((_LANES,), 0, jnp.int32) + off + r
            vs = [plsc.load_gather(s_v, [iota + c * _LANES, rcol])
                  for c in range(pchunks)]
            t = neg_fill
            for i in range(_K):
                m = [jnp.max(v) for v in vs]
                g = jnp.maximum(jnp.maximum(m[0], m[1]),
                                jnp.maximum(m[2], m[3]))
                t = jnp.where(lane_masks[i], g, t)
                taken = jnp.zeros((), jnp.bool_)
                nvs = []
                for c in range(pchunks):
                    hit = m[c] == g
                    take = jnp.logical_and(hit, jnp.logical_not(taken))
                    taken = jnp.logical_or(taken, hit)
                    lane = jnp.min(jnp.where(vs[c] == g, iota, 99))
                    rm = jnp.logical_and(iota == lane, take)
                    nvs.append(jnp.where(rm, -3.0e38, vs[c]))
                vs = nvs
            o_v[r] = t
            return carry

        lax.fori_loop(0, rows_w, body, 0)
        pltpu.sync_copy(o_v, out_hbm.at[pl.ds(base, rows_w)])

    return k(sims, gate)


def _tc_stats(user_emb, pos_emb, interpret=False):
    B, D = user_emb.shape
    BM = 256

    def body(u_ref, p_ref, o_ref):
        i = pl.program_id(0)
        u = u_ref[...]
        p = p_ref[...]
        un = u * lax.rsqrt(jnp.maximum(jnp.sum(u * u, axis=1, keepdims=True), 1e-24))
        pn = p * lax.rsqrt(jnp.maximum(jnp.sum(p * p, axis=1, keepdims=True), 1e-24))
        s = lax.dot_general(
            un, pn, (((1,), (1,)), ((), ())),
            preferred_element_type=jnp.float32,
            precision=lax.Precision.HIGHEST) * (1.0 / _TEMP)
        rows = i * BM + lax.broadcasted_iota(jnp.int32, (BM, B), 0)
        cols = lax.broadcasted_iota(jnp.int32, (BM, B), 1)
        diag = rows == cols
        se = jnp.sum(jnp.where(diag, 0.0, jnp.exp(s)), axis=1)
        rmax = jnp.max(jnp.where(diag, -3.0e38, s), axis=1)
        pos = jnp.sum(jnp.where(diag, s, 0.0), axis=1)
        o_ref[...] = jnp.concatenate(
            [pos[:, None], se[:, None], rmax[:, None],
             jnp.zeros((BM, 5), jnp.float32)], axis=1)

    return pl.pallas_call(
        body,
        grid=(B // BM,),
        in_specs=[pl.BlockSpec((BM, D), lambda i: (i, 0)),
                  pl.BlockSpec((B, D), lambda i: (0, 0))],
        out_specs=pl.BlockSpec((BM, 8), lambda i: (i, 0)),
        out_shape=jax.ShapeDtypeStruct((B, 8), jnp.float32),
        interpret=interpret,
    )(user_emb, pos_emb)


def _tc_combine(stats, top_lo, top_hi, interpret=False):
    def body(st_ref, lo_ref, hi_ref, o_ref):
        pos = st_ref[:, 0:1]
        se = st_ref[:, 1:2]
        rmax = st_ref[:, 2:3]
        hn_lo = lo_ref[...] * (1.0 / _TEMP)
        hn_hi = hi_ref[...] * (1.0 / _TEMP)
        he = jnp.concatenate(
            [jnp.sum(jnp.exp(hn_lo), axis=1, keepdims=True),
             jnp.sum(jnp.exp(hn_hi), axis=1, keepdims=True)], axis=0)
        hmax = jnp.concatenate([hn_lo[:, 0:1], hn_hi[:, 0:1]], axis=0)
        lse = jnp.log(se + jnp.exp(pos) + he)
        o_ref[0] = jnp.mean(lse - pos)
        maxo = jnp.maximum(rmax, hmax)
        o_ref[1] = jnp.mean((pos >= maxo).astype(jnp.float32))
        o_ref[2] = jnp.mean(pos)
        o_ref[3] = jnp.mean(hmax)

    return pl.pallas_call(
        body,
        out_specs=pl.BlockSpec(memory_space=pltpu.SMEM),
        out_shape=jax.ShapeDtypeStruct((4,), jnp.float32),
        interpret=interpret,
    )(stats, top_lo, top_hi)


def kernel(user_emb, pos_emb, neg_emb_pool):
    poolT = jnp.transpose(neg_emb_pool, (1, 0, 2))
    top_hi = _sc_mine_hi(user_emb, poolT, _LO)
    sims = _tc_sims(user_emb, poolT)
    top_lo = _sc_topk(sims, lax.slice(sims, (0, 0), (8, 16)))
    stats = _tc_stats(user_emb, pos_emb)
    out = _tc_combine(stats, top_lo, top_hi)
    return (out[0], out[1], out[2], out[3])
